# baseline (speedup 1.0000x reference)
import jax
import jax.numpy as jnp
from jax import lax
from jax.experimental import pallas as pl
from jax.experimental.pallas import tpu as pltpu
from jax.experimental.pallas import tpu_sc as plsc

V = 1_000_000
D = 64
B = 16_384
NC = 2
NS = 16
L = 16

V2 = V // 2
ROWS2_PER_SC = V2 // NC
C2 = 7_680
NCHUNKS = -(-ROWS2_PER_SC // C2)
PT2 = C2 // NS
CHUNK = 2 * C2
IDX_PER_TILE = B // NS
NVEC = IDX_PER_TILE // L
TRASH = C2
CAP = IDX_PER_TILE + 2 * L
PADSLOT = CAP - 1
HALFBIT = 15


def _body(x_hbm, idx_hbm, td_hbm, out_hbm,
          my_idx, combo, pos, pos_b, slot_b, trowsP, xv, chunk_buf):
    c = lax.axis_index("c")
    s = lax.axis_index("s")
    sc_base2 = c * ROWS2_PER_SC
    lane = lax.iota(jnp.int32, L)
    zero16 = jnp.zeros((L,), jnp.float32)

    pltpu.sync_copy(idx_hbm.at[pl.ds(s * IDX_PER_TILE, IDX_PER_TILE)], my_idx)

    def chunk_body(ci, carry):
        base2 = sc_base2 + jnp.minimum(ci * C2, ROWS2_PER_SC - C2)
        base = 2 * base2

        pltpu.sync_copy(x_hbm.at[pl.ds(base2 + s * PT2, PT2)], xv)
        pltpu.sync_copy(xv, chunk_buf.at[pl.ds(s * PT2, PT2)])

        def scan_body(v, cnt):
            idx16 = my_idx[pl.ds(v * L, L)]
            rel = idx16 - base
            m = (rel >= 0) & (rel < CHUNK)
            pcnt = plsc.all_reduce_population_count(m)[0]

            def with_matches(cnt):
                mi = m.astype(jnp.int32)
                off = plsc.cumsum(mi) - mi
                dst = jnp.where(m, cnt + off, PADSLOT)
                cmb = (rel >> 1) | ((rel & 1) << HALFBIT)
                plsc.store_scatter(combo, [dst], cmb, mask=m)
                plsc.store_scatter(pos, [dst],
                                   lane + (s * IDX_PER_TILE + v * L), mask=m)
                return cnt + pcnt

            return lax.cond(pcnt > 0, with_matches, lambda cnt: cnt, cnt)

        cnt = lax.fori_loop(0, NVEC, scan_body, jnp.int32(0))
        combo[pl.ds(cnt, L)] = jnp.full((L,), TRASH, jnp.int32)
        pos[pl.ds(cnt, L)] = jnp.zeros((L,), jnp.int32)

        plsc.subcore_barrier()

        def b_body(b, carry2):
            cmb16 = combo[pl.ds(b * L, L)]
            slot_b[...] = cmb16 & ((1 << HALFBIT) - 1)
            pos_b[...] = pos[pl.ds(b * L, L)]
            half = (cmb16 >> HALFBIT) & 1
            pltpu.sync_copy(td_hbm.at[pos_b], trowsP)
            othercol = (1 - half) * D
            for k in range(D):
                plsc.store_scatter(trowsP, [lane, othercol + k], zero16)
            pltpu.sync_copy(trowsP, chunk_buf.at[slot_b], add=True)
            return carry2

        lax.fori_loop(0, (cnt + (L - 1)) // L, b_body, jnp.int32(0))

        plsc.subcore_barrier()
        pltpu.sync_copy(chunk_buf.at[pl.ds(s * PT2, PT2)], xv)
        pltpu.sync_copy(xv, out_hbm.at[pl.ds(base2 + s * PT2, PT2)])
        return carry

    lax.fori_loop(0, NCHUNKS, chunk_body, jnp.int32(0))


@jax.jit
def _index_add(x2, idx32, tdup):
    mesh = plsc.VectorSubcoreMesh(core_axis_name="c", subcore_axis_name="s")
    f = pl.kernel(
        _body,
        out_type=jax.ShapeDtypeStruct((V2, 2 * D), jnp.float32),
        mesh=mesh,
        scratch_types=[
            pltpu.VMEM((IDX_PER_TILE,), jnp.int32),
            pltpu.VMEM((CAP,), jnp.int32),
            pltpu.VMEM((CAP,), jnp.int32),
            pltpu.VMEM((L,), jnp.int32),
            pltpu.VMEM((L,), jnp.int32),
            pltpu.VMEM((L, 2 * D), jnp.float32),
            pltpu.VMEM((PT2, 2 * D), jnp.float32),
            pltpu.VMEM_SHARED((C2 + 8, 2 * D), jnp.float32),
        ],
        compiler_params=pltpu.CompilerParams(needs_layout_passes=False),
    )
    return f(x2, idx32, tdup)


def kernel(x, dim, index, t):
    idx32 = (index + dim).astype(jnp.int32)
    x2 = x.reshape(V2, 2 * D)
    tdup = jnp.concatenate([t, t], axis=1)
    out2 = _index_add(x2, idx32, tdup)
    return out2.reshape(V, D)


# speedup vs baseline: 1.0234x; 1.0234x over previous
import jax
import jax.numpy as jnp
from jax import lax
from jax.experimental import pallas as pl
from jax.experimental.pallas import tpu as pltpu
from jax.experimental.pallas import tpu_sc as plsc

V = 1_000_000
D = 64
B = 16_384
NC = 2
NS = 16
L = 16

V2 = V // 2
ROWS2_PER_SC = V2 // NC
C2 = 7_680
NCHUNKS = -(-ROWS2_PER_SC // C2)
PT2 = C2 // NS
CHUNK = 2 * C2
IDX_PER_TILE = B // NS
NVEC = IDX_PER_TILE // L
TRASH = C2
CAP = IDX_PER_TILE + 2 * L
PADSLOT = CAP - 1
HALFBIT = 15


def _body(x_hbm, idx_hbm, td_hbm, out_hbm,
          my_idx, combo, pos, pos_b, slot_b, trowsP, xv, chunk_buf):
    c = lax.axis_index("c")
    s = lax.axis_index("s")
    sc_base2 = c * ROWS2_PER_SC
    lane = lax.iota(jnp.int32, L)
    zero16 = jnp.zeros((L,), jnp.float32)

    pltpu.sync_copy(idx_hbm.at[pl.ds(s * IDX_PER_TILE, IDX_PER_TILE)], my_idx)

    def chunk_body(ci, carry):
        base2 = sc_base2 + jnp.minimum(ci * C2, ROWS2_PER_SC - C2)
        base = 2 * base2

        pltpu.sync_copy(x_hbm.at[pl.ds(base2 + s * PT2, PT2)],
                        chunk_buf.at[pl.ds(s * PT2, PT2)])

        def scan_body(v, cnt):
            idx16 = my_idx[pl.ds(v * L, L)]
            rel = idx16 - base
            m = (rel >= 0) & (rel < CHUNK)
            pcnt = plsc.all_reduce_population_count(m)[0]

            def with_matches(cnt):
                mi = m.astype(jnp.int32)
                off = plsc.cumsum(mi) - mi
                dst = jnp.where(m, cnt + off, PADSLOT)
                cmb = (rel >> 1) | ((rel & 1) << HALFBIT)
                plsc.store_scatter(combo, [dst], cmb, mask=m)
                plsc.store_scatter(pos, [dst],
                                   lane + (s * IDX_PER_TILE + v * L), mask=m)
                return cnt + pcnt

            return lax.cond(pcnt > 0, with_matches, lambda cnt: cnt, cnt)

        cnt = lax.fori_loop(0, NVEC, scan_body, jnp.int32(0))
        combo[pl.ds(cnt, L)] = jnp.full((L,), TRASH, jnp.int32)
        pos[pl.ds(cnt, L)] = jnp.zeros((L,), jnp.int32)

        plsc.subcore_barrier()

        def b_body(b, carry2):
            cmb16 = combo[pl.ds(b * L, L)]
            slot_b[...] = cmb16 & ((1 << HALFBIT) - 1)
            pos_b[...] = pos[pl.ds(b * L, L)]
            half = (cmb16 >> HALFBIT) & 1
            pltpu.sync_copy(td_hbm.at[pos_b], trowsP)
            othercol = (1 - half) * D
            for k in range(D):
                plsc.store_scatter(trowsP, [lane, othercol + k], zero16)
            pltpu.sync_copy(trowsP, chunk_buf.at[slot_b], add=True)
            return carry2

        lax.fori_loop(0, (cnt + (L - 1)) // L, b_body, jnp.int32(0))

        plsc.subcore_barrier()
        pltpu.sync_copy(chunk_buf.at[pl.ds(s * PT2, PT2)],
                        out_hbm.at[pl.ds(base2 + s * PT2, PT2)])
        plsc.subcore_barrier()
        return carry

    lax.fori_loop(0, NCHUNKS, chunk_body, jnp.int32(0))


@jax.jit
def _index_add(x2, idx32, tdup):
    mesh = plsc.VectorSubcoreMesh(core_axis_name="c", subcore_axis_name="s")
    f = pl.kernel(
        _body,
        out_type=jax.ShapeDtypeStruct((V2, 2 * D), jnp.float32),
        mesh=mesh,
        scratch_types=[
            pltpu.VMEM((IDX_PER_TILE,), jnp.int32),
            pltpu.VMEM((CAP,), jnp.int32),
            pltpu.VMEM((CAP,), jnp.int32),
            pltpu.VMEM((L,), jnp.int32),
            pltpu.VMEM((L,), jnp.int32),
            pltpu.VMEM((L, 2 * D), jnp.float32),
            pltpu.VMEM((PT2, 2 * D), jnp.float32),
            pltpu.VMEM_SHARED((C2 + 8, 2 * D), jnp.float32),
        ],
        compiler_params=pltpu.CompilerParams(needs_layout_passes=False),
    )
    return f(x2, idx32, tdup)


def kernel(x, dim, index, t):
    idx32 = (index + dim).astype(jnp.int32)
    x2 = x.reshape(V2, 2 * D)
    tdup = jnp.concatenate([t, t], axis=1)
    out2 = _index_add(x2, idx32, tdup)
    return out2.reshape(V, D)
